# baseline (speedup 1.0000x reference)
import functools

import jax
import jax.numpy as jnp
from jax import lax
from jax.experimental import pallas as pl
from jax.experimental.pallas import tpu as pltpu, tpu_sc as plsc

NC = 2
NS = 16
EC = 128


def _partition_kernel(Nf, prows, cap, half, nreal):
    NW = NC * NS
    mesh = plsc.VectorSubcoreMesh(core_axis_name="c", subcore_axis_name="s",
                                  num_cores=NC, num_subcores=NS)

    out_types = (
        jax.ShapeDtypeStruct((NW, Nf), jnp.float32),
        jax.ShapeDtypeStruct((NW, 4, cap), jnp.int32),
        jax.ShapeDtypeStruct((NW, 16), jnp.int32),
    )
    scratch = [pltpu.VMEM((prows, 128), jnp.int32),
               pltpu.VMEM((Nf,), jnp.float32)]
    scratch += [pltpu.VMEM((cap,), jnp.int32) for _ in range(4)]
    scratch += [pltpu.VMEM((16,), jnp.int32)]

    @functools.partial(
        pl.kernel, out_type=out_types, mesh=mesh, scratch_types=scratch,
        compiler_params=pltpu.CompilerParams(needs_layout_passes=False),
    )
    def k(packed_hbm, zeros_hbm, dummy_hbm, hist_out, lists_out, cnt_out,
          pbuf, hist, q0, q1, q2, q3, cbuf):
        c = lax.axis_index("c")
        s = lax.axis_index("s")
        wid = s * NC + c
        qbufs = (q0, q1, q2, q3)
        pltpu.sync_copy(zeros_hbm, hist)
        pltpu.sync_copy(packed_hbm.at[wid], pbuf)
        for q in range(4):
            pltpu.sync_copy(dummy_hbm.at[q], qbufs[q])
        ones = jnp.full((16,), 1.0, jnp.float32)

        def body(j, pos):
            p0, p1, p2, p3 = pos
            for o in range(128 // 16):
                v = pbuf[j, pl.ds(o * 16, 16)]
                row = lax.shift_right_logical(v, 14)
                col = lax.bitwise_and(v, 16383)
                plsc.addupdate_scatter(hist, [row], ones)
                rhi = row >= half
                chi = col >= half
                rlo = row < half
                val = jnp.logical_and(rhi, row < nreal)
                m0 = jnp.logical_and(rlo, jnp.logical_not(chi))
                m1 = jnp.logical_and(rlo, chi)
                m2 = jnp.logical_and(val, jnp.logical_not(chi))
                m3 = jnp.logical_and(val, chi)
                plsc.store_compressed(q0.at[pl.ds(p0, 16)], v, mask=m0)
                plsc.store_compressed(q1.at[pl.ds(p1, 16)], v, mask=m1)
                plsc.store_compressed(q2.at[pl.ds(p2, 16)], v, mask=m2)
                plsc.store_compressed(q3.at[pl.ds(p3, 16)], v, mask=m3)
                p0 = p0 + jnp.sum(m0.astype(jnp.int32))
                p1 = p1 + jnp.sum(m1.astype(jnp.int32))
                p2 = p2 + jnp.sum(m2.astype(jnp.int32))
                p3 = p3 + jnp.sum(m3.astype(jnp.int32))
            return (p0, p1, p2, p3)

        z = jnp.int32(0)
        pos = lax.fori_loop(0, prows, body, (z, z, z, z))
        for q in range(4):
            pltpu.sync_copy(qbufs[q], lists_out.at[wid, q])
        io16 = lax.iota(jnp.int32, 16)
        cv = jnp.where(io16 == 0, pos[0],
             jnp.where(io16 == 1, pos[1],
             jnp.where(io16 == 2, pos[2],
             jnp.where(io16 == 3, pos[3], 0))))
        cbuf[...] = cv
        pltpu.sync_copy(cbuf, cnt_out.at[wid])
        pltpu.sync_copy(hist, hist_out.at[wid])

    return k


def _aggr_kernel(D, cap, half, HR):
    rp = HR // NS
    rg = half // NS
    mesh = plsc.VectorSubcoreMesh(core_axis_name="c", subcore_axis_name="s",
                                  num_cores=NC, num_subcores=NS)

    scratch = [
        pltpu.VMEM((EC,), jnp.int32),
        pltpu.VMEM((EC,), jnp.int32),
        pltpu.VMEM((EC,), jnp.int32),
        pltpu.VMEM((EC,), jnp.int32),
        pltpu.VMEM((EC, D), jnp.float32),
        pltpu.VMEM((16,), jnp.int32),
        pltpu.VMEM((16,), jnp.int32),
        pltpu.SemaphoreType.DMA,
        pltpu.SemaphoreType.DMA,
        pltpu.SemaphoreType.DMA,
        pltpu.VMEM_SHARED((half, D), jnp.float32),
        pltpu.VMEM_SHARED((HR, D), jnp.float32),
    ]

    @functools.partial(
        pl.kernel,
        out_type=jax.ShapeDtypeStruct((NC, HR, D), jnp.float32),
        mesh=mesh, scratch_types=scratch,
        compiler_params=pltpu.CompilerParams(needs_layout_passes=False),
    )
    def k(lists_hbm, cnts_hbm, g_hbm, zeros_hbm, out_hbm,
          lbA, lbB, colv, rowv, gbuf, cb0, cb1, sem, semA, semB, g_sh, acc):
        c = lax.axis_index("c")
        s = lax.axis_index("s")
        pltpu.sync_copy(zeros_hbm, acc.at[pl.ds(s * rp, rp)])
        pltpu.sync_copy(cnts_hbm.at[2 * s], cb0)
        pltpu.sync_copy(cnts_hbm.at[2 * s + 1], cb1)
        io16 = lax.iota(jnp.int32, 16)
        cv0 = cb0[...]
        cv1 = cb1[...]

        def stage(ch):
            pltpu.sync_copy(g_hbm.at[pl.ds(ch * half + s * rg, rg)],
                            g_sh.at[pl.ds(s * rg, rg)])

        def process(k0t, q, cnt, col_base):
            nch = lax.div(cnt + (EC - 1), jnp.int32(EC))

            def fetch(j, lb, sm):
                pltpu.async_copy(lists_hbm.at[k0t, q, pl.ds(j * EC, EC)],
                                 lb, sm)

            def consume(j, lb, sm):
                pltpu.make_async_copy(
                    lists_hbm.at[k0t, q, pl.ds(j * EC, EC)], lb, sm).wait()
                for o in range(EC // 16):
                    v = lb[pl.ds(o * 16, 16)]
                    colv[pl.ds(o * 16, 16)] = \
                        lax.bitwise_and(v, 16383) - col_base
                    rowv[pl.ds(o * 16, 16)] = \
                        lax.shift_right_logical(v, 14) - c * half

            @pl.when(nch > 0)
            def _():
                fetch(0, lbA, semA)

            @pl.when(nch > 1)
            def _():
                fetch(1, lbB, semB)

            def body(p, carry):
                j = 2 * p
                consume(j, lbA, semA)

                @pl.when(j + 2 < nch)
                def _():
                    fetch(j + 2, lbA, semA)

                pltpu.async_copy(g_sh.at[colv], gbuf, sem).wait()
                pltpu.sync_copy(gbuf, acc.at[rowv], add=True)

                @pl.when(j + 1 < nch)
                def _():
                    consume(j + 1, lbB, semB)

                    @pl.when(j + 3 < nch)
                    def _():
                        fetch(j + 3, lbB, semB)

                    pltpu.async_copy(g_sh.at[colv], gbuf, sem).wait()
                    pltpu.sync_copy(gbuf, acc.at[rowv], add=True)

                return carry

            lax.fori_loop(0, lax.div(nch + 1, jnp.int32(2)), body, 0)

        def cnt_of(cv, q):
            return jnp.sum(jnp.where(io16 == q, cv, 0))

        stage(c)
        plsc.subcore_barrier()
        qa = 3 * c
        process(2 * s, qa, cnt_of(cv0, qa), c * half)
        process(2 * s + 1, qa, cnt_of(cv1, qa), c * half)
        plsc.subcore_barrier()
        stage(1 - c)
        plsc.subcore_barrier()
        qb = 2 * c + (1 - c)
        process(2 * s, qb, cnt_of(cv0, qb), (1 - c) * half)
        process(2 * s + 1, qb, cnt_of(cv1, qb), (1 - c) * half)
        plsc.subcore_barrier()
        pltpu.sync_copy(acc.at[pl.ds(s * rp, rp)],
                        out_hbm.at[c, pl.ds(s * rp, rp)])

    return k


def _linear_kernel(x_ref, w_ref, b_ref, p_ref, g_ref):
    deg = jnp.sum(p_ref[...], axis=0) + 1.0
    dsq = lax.rsqrt(deg)[:, None]
    h = lax.dot_general(x_ref[...], w_ref[...],
                        (((1,), (1,)), ((), ())),
                        preferred_element_type=jnp.float32) + b_ref[...]
    g_ref[...] = h * dsq


def _final_kernel(g_ref, s_ref, p_ref, o_ref):
    deg = jnp.sum(p_ref[...], axis=0) + 1.0
    dsq = lax.rsqrt(deg)[:, None]
    o_ref[...] = dsq * (g_ref[...] + s_ref[0])


def kernel(x, edge_index, W, b):
    N, Din = x.shape
    Dout = W.shape[0]
    E = edge_index.shape[1]
    NW = NC * NS
    NB = 512
    half = NB * (-(-N // (2 * NB)))
    HR = half + NB

    rows = edge_index[0].astype(jnp.int32)
    cols = edge_index[1].astype(jnp.int32)
    ept = 128 * (-(-E // (128 * NW)))
    prows = ept // 128
    cap = ept
    pad = ept * NW - E
    rows_pad = jnp.full((pad,), N, jnp.int32)
    cols_pad = jnp.zeros((pad,), jnp.int32)
    packed = (jnp.concatenate([rows, rows_pad]) * 16384
              + jnp.concatenate([cols, cols_pad]))
    packed_p = packed.reshape(NW, prows, 128)
    dummies = jnp.array(
        [half * 16384 + 0, half * 16384 + half,
         N * 16384 + 0, N * 16384 + half], jnp.int32)
    dummy_hbm = jnp.broadcast_to(dummies[:, None], (4, cap))
    Nf = 128 * (-(-(N + 1) // 128))
    zerosF = jnp.zeros((Nf,), jnp.float32)
    zerosR = jnp.zeros((HR // NS, Dout), jnp.float32)
    assert half % NB == 0 and HR % (16 * NS) == 0 and half % (16 * NS) == 0
    b2 = b.reshape(1, Dout)

    hists, lists, cnts = _partition_kernel(Nf, prows, cap, half, N)(
        packed_p, zerosF, dummy_hbm)

    nblk = -(-N // NB)
    Ng = NB * nblk
    g = pl.pallas_call(
        _linear_kernel,
        grid=(nblk,),
        in_specs=[
            pl.BlockSpec((NB, Din), lambda i: (i, 0)),
            pl.BlockSpec((Dout, Din), lambda i: (0, 0)),
            pl.BlockSpec((1, Dout), lambda i: (0, 0)),
            pl.BlockSpec((NW, NB), lambda i: (0, i)),
        ],
        out_specs=pl.BlockSpec((NB, Dout), lambda i: (i, 0)),
        out_shape=jax.ShapeDtypeStruct((Ng, Dout), jnp.float32),
    )(x, W, b2, hists)

    sp = _aggr_kernel(Dout, cap, half, HR)(lists, cnts, g, zerosR)

    blocks_per_half = half // NB
    out = pl.pallas_call(
        _final_kernel,
        grid=(nblk,),
        in_specs=[
            pl.BlockSpec((NB, Dout), lambda i: (i, 0)),
            pl.BlockSpec((1, NB, Dout),
                         lambda i: (i // blocks_per_half,
                                    i % blocks_per_half, 0)),
            pl.BlockSpec((NW, NB), lambda i: (0, i)),
        ],
        out_specs=pl.BlockSpec((NB, Dout), lambda i: (i, 0)),
        out_shape=jax.ShapeDtypeStruct((N, Dout), jnp.float32),
    )(g, sp, hists)
    return out


# speedup vs baseline: 1.0199x; 1.0199x over previous
import functools

import jax
import jax.numpy as jnp
from jax import lax
from jax.experimental import pallas as pl
from jax.experimental.pallas import tpu as pltpu, tpu_sc as plsc

NC = 2
NS = 16
EC = 128


def _partition_kernel(Nf, prows, cap, half, nreal):
    NW = NC * NS
    mesh = plsc.VectorSubcoreMesh(core_axis_name="c", subcore_axis_name="s",
                                  num_cores=NC, num_subcores=NS)

    out_types = (
        jax.ShapeDtypeStruct((NW, Nf), jnp.float32),
        jax.ShapeDtypeStruct((NW, 4, cap), jnp.int32),
        jax.ShapeDtypeStruct((NW, 16), jnp.int32),
    )
    scratch = [pltpu.VMEM((prows, 128), jnp.int32),
               pltpu.VMEM((Nf,), jnp.float32)]
    scratch += [pltpu.VMEM((cap,), jnp.int32) for _ in range(4)]
    scratch += [pltpu.VMEM((16,), jnp.int32)]

    @functools.partial(
        pl.kernel, out_type=out_types, mesh=mesh, scratch_types=scratch,
        compiler_params=pltpu.CompilerParams(needs_layout_passes=False),
    )
    def k(packed_hbm, zeros_hbm, dummy_hbm, hist_out, lists_out, cnt_out,
          pbuf, hist, q0, q1, q2, q3, cbuf):
        c = lax.axis_index("c")
        s = lax.axis_index("s")
        wid = s * NC + c
        qbufs = (q0, q1, q2, q3)
        pltpu.sync_copy(zeros_hbm, hist)
        pltpu.sync_copy(packed_hbm.at[wid], pbuf)
        for q in range(4):
            pltpu.sync_copy(dummy_hbm.at[q], qbufs[q])
        ones = jnp.full((16,), 1.0, jnp.float32)

        def body(j, pos):
            p0, p1, p2, p3 = pos
            for o in range(128 // 16):
                v = pbuf[j, pl.ds(o * 16, 16)]
                row = lax.shift_right_logical(v, 14)
                col = lax.bitwise_and(v, 16383)
                plsc.addupdate_scatter(hist, [row], ones)
                rhi = row >= half
                chi = col >= half
                rlo = row < half
                val = jnp.logical_and(rhi, row < nreal)
                m0 = jnp.logical_and(rlo, jnp.logical_not(chi))
                m1 = jnp.logical_and(rlo, chi)
                m2 = jnp.logical_and(val, jnp.logical_not(chi))
                m3 = jnp.logical_and(val, chi)
                plsc.store_compressed(q0.at[pl.ds(p0, 16)], v, mask=m0)
                plsc.store_compressed(q1.at[pl.ds(p1, 16)], v, mask=m1)
                plsc.store_compressed(q2.at[pl.ds(p2, 16)], v, mask=m2)
                plsc.store_compressed(q3.at[pl.ds(p3, 16)], v, mask=m3)
                p0 = p0 + jnp.sum(m0.astype(jnp.int32))
                p1 = p1 + jnp.sum(m1.astype(jnp.int32))
                p2 = p2 + jnp.sum(m2.astype(jnp.int32))
                p3 = p3 + jnp.sum(m3.astype(jnp.int32))
            return (p0, p1, p2, p3)

        z = jnp.int32(0)
        pos = lax.fori_loop(0, prows, body, (z, z, z, z))
        for q in range(4):
            pltpu.sync_copy(qbufs[q], lists_out.at[wid, q])
        io16 = lax.iota(jnp.int32, 16)
        cv = jnp.where(io16 == 0, pos[0],
             jnp.where(io16 == 1, pos[1],
             jnp.where(io16 == 2, pos[2],
             jnp.where(io16 == 3, pos[3], 0))))
        cbuf[...] = cv
        pltpu.sync_copy(cbuf, cnt_out.at[wid])
        pltpu.sync_copy(hist, hist_out.at[wid])

    return k


def _aggr_kernel(D, cap, half, HR):
    rp = HR // NS
    rg = half // NS
    mesh = plsc.VectorSubcoreMesh(core_axis_name="c", subcore_axis_name="s",
                                  num_cores=NC, num_subcores=NS)

    scratch = [
        pltpu.VMEM((EC,), jnp.int32),
        pltpu.VMEM((EC,), jnp.int32),
        pltpu.VMEM((EC,), jnp.int32),
        pltpu.VMEM((EC,), jnp.int32),
        pltpu.VMEM((EC, D), jnp.float32),
        pltpu.VMEM((16,), jnp.int32),
        pltpu.VMEM((16,), jnp.int32),
        pltpu.SemaphoreType.DMA,
        pltpu.SemaphoreType.DMA,
        pltpu.SemaphoreType.DMA,
        pltpu.VMEM_SHARED((half, D), jnp.float32),
        pltpu.VMEM_SHARED((HR, D), jnp.float32),
    ]

    @functools.partial(
        pl.kernel,
        out_type=jax.ShapeDtypeStruct((NC, HR, D), jnp.float32),
        mesh=mesh, scratch_types=scratch,
        compiler_params=pltpu.CompilerParams(needs_layout_passes=False),
    )
    def k(lists_hbm, cnts_hbm, g_hbm, zeros_hbm, out_hbm,
          lbA, lbB, colv, rowv, gbuf, cb0, cb1, sem, semA, semB, g_sh, acc):
        c = lax.axis_index("c")
        s = lax.axis_index("s")
        pltpu.sync_copy(zeros_hbm, acc.at[pl.ds(s * rp, rp)])
        pltpu.sync_copy(cnts_hbm.at[2 * s], cb0)
        pltpu.sync_copy(cnts_hbm.at[2 * s + 1], cb1)
        io16 = lax.iota(jnp.int32, 16)
        cv0 = cb0[...]
        cv1 = cb1[...]

        def stage(ch):
            pltpu.sync_copy(g_hbm.at[pl.ds(ch * half + s * rg, rg)],
                            g_sh.at[pl.ds(s * rg, rg)])

        def process(k0t, q, cnt, col_base):
            nch = lax.div(cnt + (EC - 1), jnp.int32(EC))

            def body(j, carry):
                pltpu.sync_copy(lists_hbm.at[k0t, q, pl.ds(j * EC, EC)], lbA)
                for o in range(EC // 16):
                    v = lbA[pl.ds(o * 16, 16)]
                    colv[pl.ds(o * 16, 16)] = \
                        lax.bitwise_and(v, 16383) - col_base
                    rowv[pl.ds(o * 16, 16)] = \
                        lax.shift_right_logical(v, 14) - c * half
                pltpu.async_copy(g_sh.at[colv], gbuf, sem).wait()
                pltpu.sync_copy(gbuf, acc.at[rowv], add=True)
                return carry

            lax.fori_loop(0, nch, body, 0)

        def cnt_of(cv, q):
            return jnp.sum(jnp.where(io16 == q, cv, 0))

        stage(c)
        plsc.subcore_barrier()
        qa = 3 * c
        process(2 * s, qa, cnt_of(cv0, qa), c * half)
        process(2 * s + 1, qa, cnt_of(cv1, qa), c * half)
        plsc.subcore_barrier()
        stage(1 - c)
        plsc.subcore_barrier()
        qb = 2 * c + (1 - c)
        process(2 * s, qb, cnt_of(cv0, qb), (1 - c) * half)
        process(2 * s + 1, qb, cnt_of(cv1, qb), (1 - c) * half)
        plsc.subcore_barrier()
        pltpu.sync_copy(acc.at[pl.ds(s * rp, rp)],
                        out_hbm.at[c, pl.ds(s * rp, rp)])

    return k


def _linear_kernel(x_ref, w_ref, b_ref, p_ref, g_ref):
    deg = jnp.sum(p_ref[...], axis=0) + 1.0
    dsq = lax.rsqrt(deg)[:, None]
    h = lax.dot_general(x_ref[...], w_ref[...],
                        (((1,), (1,)), ((), ())),
                        preferred_element_type=jnp.float32) + b_ref[...]
    g_ref[...] = h * dsq


def _final_kernel(g_ref, s_ref, p_ref, o_ref):
    deg = jnp.sum(p_ref[...], axis=0) + 1.0
    dsq = lax.rsqrt(deg)[:, None]
    o_ref[...] = dsq * (g_ref[...] + s_ref[0])


def kernel(x, edge_index, W, b):
    N, Din = x.shape
    Dout = W.shape[0]
    E = edge_index.shape[1]
    NW = NC * NS
    NB = 512
    half = NB * (-(-N // (2 * NB)))
    HR = half + NB

    rows = edge_index[0].astype(jnp.int32)
    cols = edge_index[1].astype(jnp.int32)
    ept = 128 * (-(-E // (128 * NW)))
    prows = ept // 128
    cap = ept
    pad = ept * NW - E
    rows_pad = jnp.full((pad,), N, jnp.int32)
    cols_pad = jnp.zeros((pad,), jnp.int32)
    packed = (jnp.concatenate([rows, rows_pad]) * 16384
              + jnp.concatenate([cols, cols_pad]))
    packed_p = packed.reshape(NW, prows, 128)
    dummies = jnp.array(
        [half * 16384 + 0, half * 16384 + half,
         N * 16384 + 0, N * 16384 + half], jnp.int32)
    dummy_hbm = jnp.broadcast_to(dummies[:, None], (4, cap))
    Nf = 128 * (-(-(N + 1) // 128))
    zerosF = jnp.zeros((Nf,), jnp.float32)
    zerosR = jnp.zeros((HR // NS, Dout), jnp.float32)
    assert half % NB == 0 and HR % (16 * NS) == 0 and half % (16 * NS) == 0
    b2 = b.reshape(1, Dout)

    hists, lists, cnts = _partition_kernel(Nf, prows, cap, half, N)(
        packed_p, zerosF, dummy_hbm)

    nblk = -(-N // NB)
    Ng = NB * nblk
    g = pl.pallas_call(
        _linear_kernel,
        grid=(nblk,),
        in_specs=[
            pl.BlockSpec((NB, Din), lambda i: (i, 0)),
            pl.BlockSpec((Dout, Din), lambda i: (0, 0)),
            pl.BlockSpec((1, Dout), lambda i: (0, 0)),
            pl.BlockSpec((NW, NB), lambda i: (0, i)),
        ],
        out_specs=pl.BlockSpec((NB, Dout), lambda i: (i, 0)),
        out_shape=jax.ShapeDtypeStruct((Ng, Dout), jnp.float32),
    )(x, W, b2, hists)

    sp = _aggr_kernel(Dout, cap, half, HR)(lists, cnts, g, zerosR)

    blocks_per_half = half // NB
    out = pl.pallas_call(
        _final_kernel,
        grid=(nblk,),
        in_specs=[
            pl.BlockSpec((NB, Dout), lambda i: (i, 0)),
            pl.BlockSpec((1, NB, Dout),
                         lambda i: (i // blocks_per_half,
                                    i % blocks_per_half, 0)),
            pl.BlockSpec((NW, NB), lambda i: (0, i)),
        ],
        out_specs=pl.BlockSpec((NB, Dout), lambda i: (i, 0)),
        out_shape=jax.ShapeDtypeStruct((N, Dout), jnp.float32),
    )(g, sp, hists)
    return out
